# baseline (speedup 1.0000x reference)
import jax
import jax.numpy as jnp
from jax.experimental import pallas as pl

_TOKEN_BLOCK = 2048


def _router_kernel(x_ref, w_ref, gate_ref, i1_ref, i2_ref, logits_ref):
    x = x_ref[0]
    w = w_ref[...]
    logits = jax.lax.dot_general(
        x, w, (((1,), (1,)), ((), ())), preferred_element_type=jnp.float32)
    lt = logits.T
    logits_ref[0] = lt

    num_e = lt.shape[0]
    iota_f = jax.lax.broadcasted_iota(
        jnp.int32, lt.shape, 0).astype(jnp.float32)

    m1 = jnp.max(lt, axis=0, keepdims=True)
    i1 = jnp.min(jnp.where(lt == m1, iota_f, float(num_e)),
                 axis=0, keepdims=True)
    sel1 = iota_f == i1
    l_masked = jnp.where(sel1, -jnp.inf, lt)
    m2 = jnp.max(l_masked, axis=0, keepdims=True)
    i2 = jnp.min(jnp.where(l_masked == m2, iota_f, float(num_e)),
                 axis=0, keepdims=True)

    z = jnp.sum(jnp.exp(lt - m1), axis=0, keepdims=True)
    p1 = 1.0 / z
    p2 = jnp.exp(m2 - m1) / z
    denom = p1 + p2 + 1e-9
    w1 = p1 / denom
    w2 = p2 / denom

    gate_ref[0] = jnp.where(sel1, w1,
                            jnp.where(iota_f == i2, w2, jnp.float32(0.0)))
    i1_ref[0] = i1.astype(jnp.int32)
    i2_ref[0] = i2.astype(jnp.int32)


def kernel(hidden_states, W_gate):
    B, S, H = hidden_states.shape
    E = W_gate.shape[0]
    T = _TOKEN_BLOCK
    gate_t, i1r, i2r, logits_t = pl.pallas_call(
        _router_kernel,
        grid=(B, S // T),
        in_specs=[
            pl.BlockSpec((1, T, H), lambda b, s: (b, s, 0)),
            pl.BlockSpec((E, H), lambda b, s: (0, 0)),
        ],
        out_specs=[
            pl.BlockSpec((1, E, T), lambda b, s: (b, 0, s)),
            pl.BlockSpec((1, 1, T), lambda b, s: (b, 0, s)),
            pl.BlockSpec((1, 1, T), lambda b, s: (b, 0, s)),
            pl.BlockSpec((1, E, T), lambda b, s: (b, 0, s)),
        ],
        out_shape=[
            jax.ShapeDtypeStruct((B, E, S), jnp.float32),
            jax.ShapeDtypeStruct((B, 1, S), jnp.int32),
            jax.ShapeDtypeStruct((B, 1, S), jnp.int32),
            jax.ShapeDtypeStruct((B, E, S), jnp.float32),
        ],
    )(hidden_states, W_gate)
    gate = jnp.transpose(gate_t, (0, 2, 1))
    logits = jnp.transpose(logits_t, (0, 2, 1))
    idx = jnp.stack([i1r[:, 0, :], i2r[:, 0, :]], axis=-1)
    return (gate, idx, logits)


# speedup vs baseline: 1.0446x; 1.0446x over previous
import jax
import jax.numpy as jnp
from jax.experimental import pallas as pl

_TOKEN_BLOCK = 4096


def _router_kernel(x_ref, w_ref, gate_ref, i1_ref, i2_ref, logits_ref):
    x = x_ref[0]
    w = w_ref[...]
    logits = jax.lax.dot_general(
        x, w, (((1,), (1,)), ((), ())), preferred_element_type=jnp.float32)
    lt = logits.T
    logits_ref[0] = lt

    num_e = lt.shape[0]
    iota_f = jax.lax.broadcasted_iota(
        jnp.int32, lt.shape, 0).astype(jnp.float32)

    m1 = jnp.max(lt, axis=0, keepdims=True)
    i1 = jnp.min(jnp.where(lt == m1, iota_f, float(num_e)),
                 axis=0, keepdims=True)
    sel1 = iota_f == i1
    l_masked = jnp.where(sel1, -jnp.inf, lt)
    m2 = jnp.max(l_masked, axis=0, keepdims=True)
    i2 = jnp.min(jnp.where(l_masked == m2, iota_f, float(num_e)),
                 axis=0, keepdims=True)

    z = jnp.sum(jnp.exp(lt - m1), axis=0, keepdims=True)
    p1 = 1.0 / z
    p2 = jnp.exp(m2 - m1) / z
    denom = p1 + p2 + 1e-9
    w1 = p1 / denom
    w2 = p2 / denom

    gate_ref[0] = jnp.where(sel1, w1,
                            jnp.where(iota_f == i2, w2, jnp.float32(0.0)))
    i1_ref[0] = i1.astype(jnp.int32)
    i2_ref[0] = i2.astype(jnp.int32)


def kernel(hidden_states, W_gate):
    B, S, H = hidden_states.shape
    E = W_gate.shape[0]
    T = _TOKEN_BLOCK
    gate_t, i1r, i2r, logits_t = pl.pallas_call(
        _router_kernel,
        grid=(B, S // T),
        in_specs=[
            pl.BlockSpec((1, T, H), lambda b, s: (b, s, 0)),
            pl.BlockSpec((E, H), lambda b, s: (0, 0)),
        ],
        out_specs=[
            pl.BlockSpec((1, E, T), lambda b, s: (b, 0, s)),
            pl.BlockSpec((1, 1, T), lambda b, s: (b, 0, s)),
            pl.BlockSpec((1, 1, T), lambda b, s: (b, 0, s)),
            pl.BlockSpec((1, E, T), lambda b, s: (b, 0, s)),
        ],
        out_shape=[
            jax.ShapeDtypeStruct((B, E, S), jnp.float32),
            jax.ShapeDtypeStruct((B, 1, S), jnp.int32),
            jax.ShapeDtypeStruct((B, 1, S), jnp.int32),
            jax.ShapeDtypeStruct((B, E, S), jnp.float32),
        ],
    )(hidden_states, W_gate)
    gate = jnp.transpose(gate_t, (0, 2, 1))
    logits = jnp.transpose(logits_t, (0, 2, 1))
    idx = jnp.stack([i1r[:, 0, :], i2r[:, 0, :]], axis=-1)
    return (gate, idx, logits)


# speedup vs baseline: 1.0883x; 1.0419x over previous
import jax
import jax.numpy as jnp
from jax.experimental import pallas as pl

_TOKEN_BLOCK = 4096


def _router_kernel(x_ref, w_ref, gate_ref, idx_ref, logits_ref):
    x = x_ref[0]
    w = w_ref[...]
    logits = jax.lax.dot_general(
        x, w, (((1,), (1,)), ((), ())), preferred_element_type=jnp.float32)
    lt = logits.T
    logits_ref[0] = lt

    num_e = lt.shape[0]
    iota_f = jax.lax.broadcasted_iota(
        jnp.int32, lt.shape, 0).astype(jnp.float32)

    m1 = jnp.max(lt, axis=0, keepdims=True)
    i1 = jnp.min(jnp.where(lt == m1, iota_f, float(num_e)),
                 axis=0, keepdims=True)
    sel1 = iota_f == i1
    l_masked = jnp.where(sel1, -jnp.inf, lt)
    m2 = jnp.max(l_masked, axis=0, keepdims=True)
    i2 = jnp.min(jnp.where(l_masked == m2, iota_f, float(num_e)),
                 axis=0, keepdims=True)

    z = jnp.sum(jnp.exp(lt - m1), axis=0, keepdims=True)
    p1 = 1.0 / z
    p2 = jnp.exp(m2 - m1) / z
    denom = p1 + p2 + 1e-9
    w1 = p1 / denom
    w2 = p2 / denom

    gate_ref[0] = jnp.where(sel1, w1,
                            jnp.where(iota_f == i2, w2, jnp.float32(0.0)))
    idx_ref[0] = jnp.concatenate([i1, i2], axis=0).astype(jnp.int32)


def kernel(hidden_states, W_gate):
    B, S, H = hidden_states.shape
    E = W_gate.shape[0]
    T = _TOKEN_BLOCK
    gate_t, idx_t, logits_t = pl.pallas_call(
        _router_kernel,
        grid=(B, S // T),
        in_specs=[
            pl.BlockSpec((1, T, H), lambda b, s: (b, s, 0)),
            pl.BlockSpec((E, H), lambda b, s: (0, 0)),
        ],
        out_specs=[
            pl.BlockSpec((1, E, T), lambda b, s: (b, 0, s)),
            pl.BlockSpec((1, 2, T), lambda b, s: (b, 0, s)),
            pl.BlockSpec((1, E, T), lambda b, s: (b, 0, s)),
        ],
        out_shape=[
            jax.ShapeDtypeStruct((B, E, S), jnp.float32),
            jax.ShapeDtypeStruct((B, 2, S), jnp.int32),
            jax.ShapeDtypeStruct((B, E, S), jnp.float32),
        ],
    )(hidden_states, W_gate)
    gate = jnp.transpose(gate_t, (0, 2, 1))
    logits = jnp.transpose(logits_t, (0, 2, 1))
    idx = jnp.transpose(idx_t, (0, 2, 1))
    return (gate, idx, logits)
